# Initial kernel scaffold; baseline (speedup 1.0000x reference)
import jax
import jax.numpy as jnp
from jax.experimental import pallas as pl


def kernel(node_feats, edge_index, W, gn_alpha, gn_gamma, gn_beta, ro_W1, ro_b1, ro_W2, ro_b2):
    raise NotImplementedError("write your pallas kernel here")


# speedup vs baseline: 5.3845x; 5.3845x over previous
import functools

import jax
import jax.numpy as jnp
from jax import lax
from jax.experimental import pallas as pl
from jax.experimental.pallas import tpu as pltpu
from jax.experimental.pallas import tpu_sc as plsc

N = 10000
E = 320000
D = 128
EPS = 1e-5

NC = 2
NS = 16
NW = NC * NS
CHUNK = 128
NCHUNKS = E // CHUNK
MAXCH = -(-NCHUNKS // NW)
NPAD = 10240
TROWS = NPAD // NS
ZROWS = 128

BN = 1000
NB = N // BN

_mesh = plsc.VectorSubcoreMesh(core_axis_name="c", subcore_axis_name="s")


def _leaky(x):
    return jnp.where(x >= 0, x, 0.01 * x)


@functools.partial(
    pl.kernel,
    out_type=jax.ShapeDtypeStruct((NC, 2, NPAD), jnp.float32),
    mesh=_mesh,
    scratch_types=[
        pltpu.VMEM((CHUNK,), jnp.int32),
        pltpu.VMEM((CHUNK,), jnp.int32),
        pltpu.VMEM((CHUNK,), jnp.float32),
        pltpu.VMEM((TROWS,), jnp.float32),
        pltpu.VMEM_SHARED((NPAD,), jnp.float32),
        pltpu.VMEM_SHARED((NPAD,), jnp.float32),
    ],
)
def _deg(src_hbm, dst_hbm, out_hbm, sidx, didx, ones_v, zvec, acc_s, acc_d):
    cid = lax.axis_index("c")
    sid = lax.axis_index("s")
    w = sid * NC + cid
    for k in range(CHUNK // 16):
        ones_v[pl.ds(k * 16, 16)] = jnp.ones((16,), jnp.float32)
    for k in range(TROWS // 16):
        zvec[pl.ds(k * 16, 16)] = jnp.zeros((16,), jnp.float32)
    pltpu.sync_copy(zvec, acc_s.at[pl.ds(sid * TROWS, TROWS)])
    pltpu.sync_copy(zvec, acc_d.at[pl.ds(sid * TROWS, TROWS)])
    plsc.subcore_barrier()

    def body(j, carry):
        c = w + j * NW

        @pl.when(c < NCHUNKS)
        def _():
            off = c * CHUNK
            pltpu.sync_copy(src_hbm.at[pl.ds(off, CHUNK)], sidx)
            pltpu.sync_copy(dst_hbm.at[pl.ds(off, CHUNK)], didx)
            pltpu.sync_copy(ones_v, acc_s.at[sidx], add=True)
            pltpu.sync_copy(ones_v, acc_d.at[didx], add=True)

        return carry

    lax.fori_loop(0, MAXCH, body, 0)
    plsc.subcore_barrier()
    pltpu.sync_copy(acc_s.at[pl.ds(sid * TROWS, TROWS)],
                    out_hbm.at[cid, 0, pl.ds(sid * TROWS, TROWS)])
    pltpu.sync_copy(acc_d.at[pl.ds(sid * TROWS, TROWS)],
                    out_hbm.at[cid, 1, pl.ds(sid * TROWS, TROWS)])


@functools.partial(
    pl.kernel,
    out_type=jax.ShapeDtypeStruct((NC, NPAD, D), jnp.float32),
    mesh=_mesh,
    scratch_types=[
        pltpu.VMEM((CHUNK,), jnp.int32),
        pltpu.VMEM((CHUNK,), jnp.int32),
        pltpu.VMEM((CHUNK, D), jnp.float32),
        pltpu.VMEM((ZROWS, D), jnp.float32),
        pltpu.VMEM_SHARED((NPAD, D), jnp.float32),
        pltpu.SemaphoreType.DMA,
    ],
)
def _segsum(h_hbm, src_hbm, dst_hbm, out_hbm, sidx, didx, rows, zbuf, acc, sem):
    cid = lax.axis_index("c")
    sid = lax.axis_index("s")
    w = sid * NC + cid

    def zrow(i, carry):
        for k in range(D // 16):
            zbuf[i, pl.ds(k * 16, 16)] = jnp.zeros((16,), jnp.float32)
        return carry

    lax.fori_loop(0, ZROWS, zrow, 0)
    for b in range(TROWS // ZROWS):
        pltpu.sync_copy(zbuf, acc.at[pl.ds(sid * TROWS + b * ZROWS, ZROWS)])
    plsc.subcore_barrier()

    def body(j, carry):
        c = w + j * NW

        @pl.when(c < NCHUNKS)
        def _():
            off = c * CHUNK
            pltpu.sync_copy(src_hbm.at[pl.ds(off, CHUNK)], sidx)
            pltpu.sync_copy(dst_hbm.at[pl.ds(off, CHUNK)], didx)
            pltpu.async_copy(h_hbm.at[sidx], rows, sem).wait()
            pltpu.sync_copy(rows, acc.at[didx], add=True)

        return carry

    lax.fori_loop(0, MAXCH, body, 0)
    plsc.subcore_barrier()
    for b in range(TROWS // ZROWS):
        r0 = sid * TROWS + b * ZROWS
        pltpu.sync_copy(acc.at[pl.ds(r0, ZROWS)],
                        out_hbm.at[cid, pl.ds(r0, ZROWS)])


def _pre_body(x_ref, dp_ref, h0_ref, ns_ref, nd_ref):
    dp = dp_ref[...]
    ns = lax.rsqrt(jnp.maximum(dp[:, 0:1] + dp[:, 2:3], 1.0))
    nd = lax.rsqrt(jnp.maximum(dp[:, 1:2] + dp[:, 3:4], 1.0))
    ns_ref[...] = ns
    nd_ref[...] = nd
    h0_ref[...] = x_ref[...] * ns


def _pre(x, dp):
    return pl.pallas_call(
        _pre_body,
        grid=(NB,),
        in_specs=[
            pl.BlockSpec((BN, D), lambda i: (i, 0)),
            pl.BlockSpec((BN, 4), lambda i: (i, 0)),
        ],
        out_specs=[
            pl.BlockSpec((BN, D), lambda i: (i, 0)),
            pl.BlockSpec((BN, 1), lambda i: (i, 0)),
            pl.BlockSpec((BN, 1), lambda i: (i, 0)),
        ],
        out_shape=[
            jax.ShapeDtypeStruct((N, D), jnp.float32),
            jax.ShapeDtypeStruct((N, 1), jnp.float32),
            jax.ShapeDtypeStruct((N, 1), jnp.float32),
        ],
    )(x, dp)


def _convA_body(mp_ref, nd_ref, w_ref, x_ref, st_ref):
    i = pl.program_id(0)
    x = (mp_ref[0] + mp_ref[1]) * nd_ref[...]
    x = jnp.dot(x, w_ref[...], preferred_element_type=jnp.float32)
    x_ref[...] = x

    @pl.when(i == 0)
    def _():
        st_ref[...] = jnp.zeros_like(st_ref)

    st_ref[0:1, :] += jnp.sum(x, axis=0, keepdims=True)
    st_ref[1:2, :] += jnp.sum(x * x, axis=0, keepdims=True)


def _convA(mp, nd, w):
    return pl.pallas_call(
        _convA_body,
        grid=(NB,),
        in_specs=[
            pl.BlockSpec((NC, BN, D), lambda i: (0, i, 0)),
            pl.BlockSpec((BN, 1), lambda i: (i, 0)),
            pl.BlockSpec((D, D), lambda i: (0, 0)),
        ],
        out_specs=[
            pl.BlockSpec((BN, D), lambda i: (i, 0)),
            pl.BlockSpec((2, D), lambda i: (0, 0)),
        ],
        out_shape=[
            jax.ShapeDtypeStruct((N, D), jnp.float32),
            jax.ShapeDtypeStruct((2, D), jnp.float32),
        ],
    )(mp, nd, w)


def _convB_body(first, last, x_ref, st_ref, ga_ref, gg_ref, gb_ref,
                w1_ref, b1_ref, pr_ref, ns_ref, *out_refs):
    i = pl.program_id(0)
    a = ga_ref[...]
    mean = st_ref[0:1, :] * (1.0 / N)
    ex2 = st_ref[1:2, :] * (1.0 / N)
    var = ex2 - mean * mean * (2.0 * a - a * a)
    scale = gg_ref[...] * lax.rsqrt(var + EPS)
    uf = _leaky((x_ref[...] - a * mean) * scale + gb_ref[...])
    if first:
        ro_in = uf
        hnext = (uf + pr_ref[...]) * ns_ref[...]
    else:
        ro_in = uf + pr_ref[...]
        hnext = ro_in * ns_ref[...]
    hh = _leaky(jnp.dot(ro_in, w1_ref[...],
                        preferred_element_type=jnp.float32) + b1_ref[...])
    if last:
        s_ref, = out_refs
    else:
        h_ref, p_ref, s_ref = out_refs
        h_ref[...] = hnext
        p_ref[...] = uf

    @pl.when(i == 0)
    def _():
        s_ref[...] = jnp.zeros_like(s_ref)

    s_ref[...] += jnp.sum(hh, axis=0, keepdims=True)


def _convB(first, last, x, st, ga, gg, gb, w1, b1, pr, ns):
    out_specs = [pl.BlockSpec((1, D), lambda i: (0, 0))]
    out_shape = [jax.ShapeDtypeStruct((1, D), jnp.float32)]
    if not last:
        out_specs = [
            pl.BlockSpec((BN, D), lambda i: (i, 0)),
            pl.BlockSpec((BN, D), lambda i: (i, 0)),
        ] + out_specs
        out_shape = [
            jax.ShapeDtypeStruct((N, D), jnp.float32),
            jax.ShapeDtypeStruct((N, D), jnp.float32),
        ] + out_shape
    return pl.pallas_call(
        functools.partial(_convB_body, first, last),
        grid=(NB,),
        in_specs=[
            pl.BlockSpec((BN, D), lambda i: (i, 0)),
            pl.BlockSpec((2, D), lambda i: (0, 0)),
            pl.BlockSpec((1, D), lambda i: (0, 0)),
            pl.BlockSpec((1, D), lambda i: (0, 0)),
            pl.BlockSpec((1, D), lambda i: (0, 0)),
            pl.BlockSpec((D, D), lambda i: (0, 0)),
            pl.BlockSpec((1, D), lambda i: (0, 0)),
            pl.BlockSpec((BN, D), lambda i: (i, 0)),
            pl.BlockSpec((BN, 1), lambda i: (i, 0)),
        ],
        out_specs=out_specs,
        out_shape=out_shape,
    )(x, st, ga, gg, gb, w1, b1, pr, ns)


def _fin_body(s_ref, w2_ref, b2_ref, o_ref):
    for l in range(3):
        o_ref[l:l + 1, :] = _leaky(
            jnp.dot(s_ref[l:l + 1, :], w2_ref[l],
                    preferred_element_type=jnp.float32)
            + b2_ref[l])


def _fin(s, w2, b2):
    return pl.pallas_call(
        _fin_body,
        out_shape=jax.ShapeDtypeStruct((3, D), jnp.float32),
    )(s, w2, b2)


def kernel(node_feats, edge_index, W, gn_alpha, gn_gamma, gn_beta,
           ro_W1, ro_b1, ro_W2, ro_b2):
    src = edge_index[0]
    dst = edge_index[1]
    deg_parts = _deg(src, dst)
    dp = deg_parts.reshape(NC * 2, NPAD).T
    h, ns, nd = _pre(node_feats, dp)

    ga = gn_alpha.reshape(3, 1, D)
    gg = gn_gamma.reshape(3, 1, D)
    gb = gn_beta.reshape(3, 1, D)
    b1 = ro_b1.reshape(3, 1, D)

    ss = []
    prev_resid = node_feats
    for l in range(3):
        mp = _segsum(h, src, dst)
        x, st = _convA(mp, nd, W[l])
        first, last = l == 0, l == 2
        outs = _convB(first, last, x, st, ga[l], gg[l], gb[l],
                      ro_W1[l], b1[l], prev_resid, ns)
        if last:
            s, = outs
        else:
            h, prev_resid, s = outs
        ss.append(s)

    ro = _fin(jnp.concatenate(ss, axis=0), ro_W2, ro_b2.reshape(3, 1, D))
    return ro.reshape(3 * D)


# speedup vs baseline: 8.8758x; 1.6484x over previous
import functools

import jax
import jax.numpy as jnp
from jax import lax
from jax.experimental import pallas as pl
from jax.experimental.pallas import tpu as pltpu
from jax.experimental.pallas import tpu_sc as plsc

N = 10000
E = 320000
D = 128
EPS = 1e-5

NC = 2
NS = 16
NW = NC * NS
CHUNK = 128
NCHUNKS = E // CHUNK
MAXCH = -(-NCHUNKS // NW)
NPAD = 10240
TROWS = NPAD // NS
ZROWS = 128
ZB = 32

BN = 1000
NB = N // BN

_mesh = plsc.VectorSubcoreMesh(core_axis_name="c", subcore_axis_name="s")


def _leaky(x):
    return jnp.where(x >= 0, x, 0.01 * x)


@functools.partial(
    pl.kernel,
    out_type=jax.ShapeDtypeStruct((NC, 2, NPAD), jnp.float32),
    mesh=_mesh,
    scratch_types=[
        pltpu.VMEM((CHUNK,), jnp.int32),
        pltpu.VMEM((CHUNK,), jnp.int32),
        pltpu.VMEM((CHUNK,), jnp.float32),
        pltpu.VMEM((TROWS,), jnp.float32),
        pltpu.VMEM_SHARED((NPAD,), jnp.float32),
        pltpu.VMEM_SHARED((NPAD,), jnp.float32),
    ],
)
def _deg(src_hbm, dst_hbm, out_hbm, sidx, didx, ones_v, zvec, acc_s, acc_d):
    cid = lax.axis_index("c")
    sid = lax.axis_index("s")
    w = sid * NC + cid
    for k in range(CHUNK // 16):
        ones_v[pl.ds(k * 16, 16)] = jnp.ones((16,), jnp.float32)
    for k in range(TROWS // 16):
        zvec[pl.ds(k * 16, 16)] = jnp.zeros((16,), jnp.float32)
    pltpu.sync_copy(zvec, acc_s.at[pl.ds(sid * TROWS, TROWS)])
    pltpu.sync_copy(zvec, acc_d.at[pl.ds(sid * TROWS, TROWS)])
    plsc.subcore_barrier()

    def body(j, carry):
        c = w + j * NW

        @pl.when(c < NCHUNKS)
        def _():
            off = c * CHUNK
            pltpu.sync_copy(src_hbm.at[pl.ds(off, CHUNK)], sidx)
            pltpu.sync_copy(dst_hbm.at[pl.ds(off, CHUNK)], didx)
            pltpu.sync_copy(ones_v, acc_s.at[sidx], add=True)
            pltpu.sync_copy(ones_v, acc_d.at[didx], add=True)

        return carry

    lax.fori_loop(0, MAXCH, body, 0)
    plsc.subcore_barrier()
    pltpu.sync_copy(acc_s.at[pl.ds(sid * TROWS, TROWS)],
                    out_hbm.at[cid, 0, pl.ds(sid * TROWS, TROWS)])
    pltpu.sync_copy(acc_d.at[pl.ds(sid * TROWS, TROWS)],
                    out_hbm.at[cid, 1, pl.ds(sid * TROWS, TROWS)])


NBUF = 2
GITERS = -(-MAXCH // NBUF)


@functools.partial(
    pl.kernel,
    out_type=jax.ShapeDtypeStruct((NC, NPAD, D), jnp.float32),
    mesh=_mesh,
    scratch_types=(
        [pltpu.VMEM((2, CHUNK), jnp.int32)] * NBUF
        + [pltpu.VMEM((CHUNK, D), jnp.float32)] * NBUF
        + [
            pltpu.VMEM((ZB, D), jnp.float32),
            pltpu.VMEM_SHARED((NPAD, D), jnp.float32),
        ]
        + [pltpu.SemaphoreType.DMA] * NBUF
    ),
)
def _segsum(h_hbm, eidx_hbm, out_hbm, *scr):
    idxs = scr[:NBUF]
    rows = scr[NBUF:2 * NBUF]
    zbuf = scr[2 * NBUF]
    acc = scr[2 * NBUF + 1]
    sems = scr[2 * NBUF + 2:]
    cid = lax.axis_index("c")
    sid = lax.axis_index("s")
    w = sid * NC + cid

    def zrow(i, carry):
        for k in range(D // 16):
            zbuf[i, pl.ds(k * 16, 16)] = jnp.zeros((16,), jnp.float32)
        return carry

    lax.fori_loop(0, ZB, zrow, 0)
    for b in range(TROWS // ZB):
        pltpu.sync_copy(zbuf, acc.at[pl.ds(sid * TROWS + b * ZB, ZB)])
    plsc.subcore_barrier()

    for b in range(NBUF):
        pltpu.sync_copy(eidx_hbm.at[w + b * NW], idxs[b])
        pltpu.async_copy(h_hbm.at[idxs[b].at[0]], rows[b], sems[b])

    def body(g, carry):
        for b in range(NBUF):
            c = w + (g * NBUF + b) * NW

            @pl.when(c < NCHUNKS)
            def _():
                pltpu.make_async_copy(
                    h_hbm.at[idxs[b].at[0]], rows[b], sems[b]).wait()
                pltpu.sync_copy(rows[b], acc.at[idxs[b].at[1]], add=True)
                cn = c + NBUF * NW

                @pl.when(cn < NCHUNKS)
                def _():
                    pltpu.sync_copy(eidx_hbm.at[cn], idxs[b])
                    pltpu.async_copy(h_hbm.at[idxs[b].at[0]], rows[b], sems[b])

        return carry

    lax.fori_loop(0, GITERS, body, 0)
    plsc.subcore_barrier()
    for b in range(TROWS // ZROWS):
        r0 = sid * TROWS + b * ZROWS
        pltpu.sync_copy(acc.at[pl.ds(r0, ZROWS)],
                        out_hbm.at[cid, pl.ds(r0, ZROWS)])


def _pre_body(x_ref, dp_ref, h0_ref, ns_ref, nd_ref):
    dp = dp_ref[...]
    ns = lax.rsqrt(jnp.maximum(dp[:, 0:1] + dp[:, 2:3], 1.0))
    nd = lax.rsqrt(jnp.maximum(dp[:, 1:2] + dp[:, 3:4], 1.0))
    ns_ref[...] = ns
    nd_ref[...] = nd
    h0_ref[...] = x_ref[...] * ns


def _pre(x, dp):
    return pl.pallas_call(
        _pre_body,
        grid=(NB,),
        in_specs=[
            pl.BlockSpec((BN, D), lambda i: (i, 0)),
            pl.BlockSpec((BN, 4), lambda i: (i, 0)),
        ],
        out_specs=[
            pl.BlockSpec((BN, D), lambda i: (i, 0)),
            pl.BlockSpec((BN, 1), lambda i: (i, 0)),
            pl.BlockSpec((BN, 1), lambda i: (i, 0)),
        ],
        out_shape=[
            jax.ShapeDtypeStruct((N, D), jnp.float32),
            jax.ShapeDtypeStruct((N, 1), jnp.float32),
            jax.ShapeDtypeStruct((N, 1), jnp.float32),
        ],
    )(x, dp)


def _convA_body(mp_ref, nd_ref, w_ref, x_ref, st_ref):
    i = pl.program_id(0)
    x = (mp_ref[0] + mp_ref[1]) * nd_ref[...]
    x = jnp.dot(x, w_ref[...], preferred_element_type=jnp.float32)
    x_ref[...] = x

    @pl.when(i == 0)
    def _():
        st_ref[...] = jnp.zeros_like(st_ref)

    st_ref[0:1, :] += jnp.sum(x, axis=0, keepdims=True)
    st_ref[1:2, :] += jnp.sum(x * x, axis=0, keepdims=True)


def _convA(mp, nd, w):
    return pl.pallas_call(
        _convA_body,
        grid=(NB,),
        in_specs=[
            pl.BlockSpec((NC, BN, D), lambda i: (0, i, 0)),
            pl.BlockSpec((BN, 1), lambda i: (i, 0)),
            pl.BlockSpec((D, D), lambda i: (0, 0)),
        ],
        out_specs=[
            pl.BlockSpec((BN, D), lambda i: (i, 0)),
            pl.BlockSpec((2, D), lambda i: (0, 0)),
        ],
        out_shape=[
            jax.ShapeDtypeStruct((N, D), jnp.float32),
            jax.ShapeDtypeStruct((2, D), jnp.float32),
        ],
    )(mp, nd, w)


def _convB_body(first, last, x_ref, st_ref, ga_ref, gg_ref, gb_ref,
                w1_ref, b1_ref, pr_ref, ns_ref, *out_refs):
    i = pl.program_id(0)
    a = ga_ref[...]
    mean = st_ref[0:1, :] * (1.0 / N)
    ex2 = st_ref[1:2, :] * (1.0 / N)
    var = ex2 - mean * mean * (2.0 * a - a * a)
    scale = gg_ref[...] * lax.rsqrt(var + EPS)
    uf = _leaky((x_ref[...] - a * mean) * scale + gb_ref[...])
    if first:
        ro_in = uf
        hnext = (uf + pr_ref[...]) * ns_ref[...]
    else:
        ro_in = uf + pr_ref[...]
        hnext = ro_in * ns_ref[...]
    hh = _leaky(jnp.dot(ro_in, w1_ref[...],
                        preferred_element_type=jnp.float32) + b1_ref[...])
    if last:
        s_ref, = out_refs
    else:
        h_ref, p_ref, s_ref = out_refs
        h_ref[...] = hnext
        p_ref[...] = uf

    @pl.when(i == 0)
    def _():
        s_ref[...] = jnp.zeros_like(s_ref)

    s_ref[...] += jnp.sum(hh, axis=0, keepdims=True)


def _convB(first, last, x, st, ga, gg, gb, w1, b1, pr, ns):
    out_specs = [pl.BlockSpec((1, D), lambda i: (0, 0))]
    out_shape = [jax.ShapeDtypeStruct((1, D), jnp.float32)]
    if not last:
        out_specs = [
            pl.BlockSpec((BN, D), lambda i: (i, 0)),
            pl.BlockSpec((BN, D), lambda i: (i, 0)),
        ] + out_specs
        out_shape = [
            jax.ShapeDtypeStruct((N, D), jnp.float32),
            jax.ShapeDtypeStruct((N, D), jnp.float32),
        ] + out_shape
    return pl.pallas_call(
        functools.partial(_convB_body, first, last),
        grid=(NB,),
        in_specs=[
            pl.BlockSpec((BN, D), lambda i: (i, 0)),
            pl.BlockSpec((2, D), lambda i: (0, 0)),
            pl.BlockSpec((1, D), lambda i: (0, 0)),
            pl.BlockSpec((1, D), lambda i: (0, 0)),
            pl.BlockSpec((1, D), lambda i: (0, 0)),
            pl.BlockSpec((D, D), lambda i: (0, 0)),
            pl.BlockSpec((1, D), lambda i: (0, 0)),
            pl.BlockSpec((BN, D), lambda i: (i, 0)),
            pl.BlockSpec((BN, 1), lambda i: (i, 0)),
        ],
        out_specs=out_specs,
        out_shape=out_shape,
    )(x, st, ga, gg, gb, w1, b1, pr, ns)


def _fin_body(s_ref, w2_ref, b2_ref, o_ref):
    for l in range(3):
        o_ref[l:l + 1, :] = _leaky(
            jnp.dot(s_ref[l:l + 1, :], w2_ref[l],
                    preferred_element_type=jnp.float32)
            + b2_ref[l])


def _fin(s, w2, b2):
    return pl.pallas_call(
        _fin_body,
        out_shape=jax.ShapeDtypeStruct((3, D), jnp.float32),
    )(s, w2, b2)


def kernel(node_feats, edge_index, W, gn_alpha, gn_gamma, gn_beta,
           ro_W1, ro_b1, ro_W2, ro_b2):
    src = edge_index[0]
    dst = edge_index[1]
    eidx = edge_index.reshape(2, NCHUNKS, CHUNK).transpose(1, 0, 2)
    deg_parts = _deg(src, dst)
    dp = deg_parts.reshape(NC * 2, NPAD).T
    h, ns, nd = _pre(node_feats, dp)

    ga = gn_alpha.reshape(3, 1, D)
    gg = gn_gamma.reshape(3, 1, D)
    gb = gn_beta.reshape(3, 1, D)
    b1 = ro_b1.reshape(3, 1, D)

    ss = []
    prev_resid = node_feats
    for l in range(3):
        mp = _segsum(h, eidx)
        x, st = _convA(mp, nd, W[l])
        first, last = l == 0, l == 2
        outs = _convB(first, last, x, st, ga[l], gg[l], gb[l],
                      ro_W1[l], b1[l], prev_resid, ns)
        if last:
            s, = outs
        else:
            h, prev_resid, s = outs
        ss.append(s)

    ro = _fin(jnp.concatenate(ss, axis=0), ro_W2, ro_b2.reshape(3, 1, D))
    return ro.reshape(3 * D)
